# baseline (speedup 1.0000x reference)
import jax
import jax.numpy as jnp
from jax.experimental import pallas as pl
from jax.experimental.pallas import tpu as pltpu

_ROWS = 16384
_COLS = 200
_BLOCK_ROWS = 512
_GRID = _ROWS // _BLOCK_ROWS
_NBINS = 10


def _wmse_block(p_ref, t_ref, bins_ref, bw_ref, out_ref):
    p = p_ref[...]
    t = t_ref[...]
    l = (p - t) * (p - t)
    w = jnp.full_like(t, bw_ref[0])
    for j in range(1, _NBINS):
        w = jnp.where(t >= bins_ref[j], bw_ref[j], w)

    @pl.when(pl.program_id(0) == 0)
    def _init():
        out_ref[0, 0] = 0.0

    out_ref[0, 0] += jnp.sum(l * w)


def kernel(predictions, targets, bins, bin_weights):
    bw_scaled = bin_weights * (100.0 / (_ROWS * _COLS))
    out = pl.pallas_call(
        _wmse_block,
        grid=(_GRID,),
        in_specs=[
            pl.BlockSpec((_BLOCK_ROWS, _COLS), lambda i: (i, 0)),
            pl.BlockSpec((_BLOCK_ROWS, _COLS), lambda i: (i, 0)),
            pl.BlockSpec(memory_space=pltpu.SMEM),
            pl.BlockSpec(memory_space=pltpu.SMEM),
        ],
        out_specs=pl.BlockSpec((1, 1), lambda i: (0, 0), memory_space=pltpu.SMEM),
        out_shape=jax.ShapeDtypeStruct((1, 1), jnp.float32),
    )(predictions, targets, bins, bw_scaled)
    return out[0, 0]


# speedup vs baseline: 1.2382x; 1.2382x over previous
import jax
import jax.numpy as jnp
from jax.experimental import pallas as pl
from jax.experimental.pallas import tpu as pltpu

_ROWS = 16384
_COLS = 200
_BLOCK_ROWS = 4096
_GRID = _ROWS // _BLOCK_ROWS
_NBINS = 10


def _wmse_block(p_ref, t_ref, bins_ref, bw_ref, out_ref):
    p = p_ref[...]
    t = t_ref[...]
    l = (p - t) * (p - t)
    w = jnp.full_like(t, bw_ref[0])
    for j in range(1, _NBINS):
        w = jnp.where(t >= bins_ref[j], bw_ref[j], w)

    @pl.when(pl.program_id(0) == 0)
    def _init():
        out_ref[0, 0] = 0.0

    out_ref[0, 0] += jnp.sum(l * w)


def kernel(predictions, targets, bins, bin_weights):
    bw_scaled = bin_weights * (100.0 / (_ROWS * _COLS))
    out = pl.pallas_call(
        _wmse_block,
        grid=(_GRID,),
        in_specs=[
            pl.BlockSpec((_BLOCK_ROWS, _COLS), lambda i: (i, 0)),
            pl.BlockSpec((_BLOCK_ROWS, _COLS), lambda i: (i, 0)),
            pl.BlockSpec(memory_space=pltpu.SMEM),
            pl.BlockSpec(memory_space=pltpu.SMEM),
        ],
        out_specs=pl.BlockSpec((1, 1), lambda i: (0, 0), memory_space=pltpu.SMEM),
        out_shape=jax.ShapeDtypeStruct((1, 1), jnp.float32),
    )(predictions, targets, bins, bw_scaled)
    return out[0, 0]


# speedup vs baseline: 1.2426x; 1.0036x over previous
import jax
import jax.numpy as jnp
from jax.experimental import pallas as pl
from jax.experimental.pallas import tpu as pltpu

_ROWS = 16384
_COLS = 200
_BLOCK_ROWS = 1024
_STREAMS = 2
_GRID = _ROWS // (_BLOCK_ROWS * _STREAMS)
_NBINS = 10


def _weight(t, bins_ref, bw_ref):
    w = jnp.full_like(t, bw_ref[0])
    for j in range(1, _NBINS):
        w = jnp.where(t >= bins_ref[j], bw_ref[j], w)
    return w


def _wmse_block(p0_ref, p1_ref, t0_ref, t1_ref, bins_ref, bw_ref, out_ref):
    @pl.when(pl.program_id(0) == 0)
    def _init():
        out_ref[0, 0] = 0.0

    acc = 0.0
    for p_ref, t_ref in ((p0_ref, t0_ref), (p1_ref, t1_ref)):
        p = p_ref[...]
        t = t_ref[...]
        l = (p - t) * (p - t)
        acc += jnp.sum(l * _weight(t, bins_ref, bw_ref))
    out_ref[0, 0] += acc


def kernel(predictions, targets, bins, bin_weights):
    bw_scaled = bin_weights * (100.0 / (_ROWS * _COLS))
    row_spec_0 = pl.BlockSpec((_BLOCK_ROWS, _COLS), lambda i: (2 * i, 0))
    row_spec_1 = pl.BlockSpec((_BLOCK_ROWS, _COLS), lambda i: (2 * i + 1, 0))
    out = pl.pallas_call(
        _wmse_block,
        grid=(_GRID,),
        in_specs=[
            row_spec_0,
            row_spec_1,
            row_spec_0,
            row_spec_1,
            pl.BlockSpec(memory_space=pltpu.SMEM),
            pl.BlockSpec(memory_space=pltpu.SMEM),
        ],
        out_specs=pl.BlockSpec((1, 1), lambda i: (0, 0), memory_space=pltpu.SMEM),
        out_shape=jax.ShapeDtypeStruct((1, 1), jnp.float32),
    )(predictions, predictions, targets, targets, bins, bw_scaled)
    return out[0, 0]
